# baseline (speedup 1.0000x reference)
import functools
import math

import jax
import jax.numpy as jnp
from jax import lax
from jax.experimental import pallas as pl
from jax.experimental.pallas import tpu as pltpu
from jax.experimental.pallas import tpu_sc as plsc

LN2 = math.log(2.0)

_N = 100000
_E = 6400000
_D = 128

_BLK = 2048
_NBLK = 49
_TROW = 392
_TCOL = 256
_N_PAD = _TROW * _TCOL
_NC, _NS = 2, 16
_NW = _NC * _NS
_SLICE = _N_PAD // _NS
_C = 2048
_NK = _E // _C
_KPW = _NK // _NW
_REM = _NK - _KPW * _NW


def _mlp_body(x_ref, w0_ref, b0_ref, w1_ref, w2_ref, w3_ref, b3_ref,
              ang_ref, scl_ref):
    dn = (((1,), (1,)), ((), ()))
    x = x_ref[...]
    h = jnp.maximum(
        lax.dot_general(x, w0_ref[...], dn, preferred_element_type=jnp.float32)
        + b0_ref[...], 0.0)
    h = jnp.maximum(
        lax.dot_general(h, w1_ref[...], dn, preferred_element_type=jnp.float32),
        0.0)
    h = jnp.maximum(
        lax.dot_general(h, w2_ref[...], dn, preferred_element_type=jnp.float32),
        0.0)
    a = (lax.dot_general(w3_ref[...], h, dn, preferred_element_type=jnp.float32)
         + b3_ref[0])
    a8 = jnp.concatenate(jnp.split(a, 8, axis=1), axis=0)
    ang_ref[...] = a8
    scl_ref[...] = a8 * LN2


def _mlp(coeffs, W0, b0, W1, W2, W3, b3):
    full = lambda i: (0, 0)
    return pl.pallas_call(
        _mlp_body,
        grid=(_NBLK,),
        in_specs=[
            pl.BlockSpec((_BLK, _D), lambda i: (i, 0)),
            pl.BlockSpec((_D, _D), full),
            pl.BlockSpec((1, _D), full),
            pl.BlockSpec((_D, _D), full),
            pl.BlockSpec((_D, _D), full),
            pl.BlockSpec((1, _D), full),
            pl.BlockSpec(memory_space=pltpu.SMEM),
        ],
        out_specs=[
            pl.BlockSpec((8, _TCOL), lambda i: (i, 0)),
            pl.BlockSpec((8, _TCOL), lambda i: (i, 0)),
        ],
        out_shape=[
            jax.ShapeDtypeStruct((_TROW, _TCOL), jnp.float32),
            jax.ShapeDtypeStruct((_TROW, _TCOL), jnp.float32),
        ],
    )(coeffs, W0, b0.reshape(1, _D), W1, W2, W3, b3)


@functools.partial(
    pl.kernel,
    out_type=jax.ShapeDtypeStruct((_NC, _N_PAD), jnp.float32),
    mesh=plsc.VectorSubcoreMesh(core_axis_name="c", subcore_axis_name="s"),
    compiler_params=pltpu.CompilerParams(needs_layout_passes=False),
    scratch_types=[
        pltpu.VMEM((_N,), jnp.float32),
        pltpu.VMEM((2, _C), jnp.int32),
        pltpu.VMEM((2, _C), jnp.int32),
        pltpu.VMEM((2, _C), jnp.int32),
        pltpu.VMEM((2, _C), jnp.int32),
        pltpu.VMEM((_C,), jnp.int32),
        pltpu.VMEM((_C,), jnp.int32),
        pltpu.VMEM((_C,), jnp.float32),
        pltpu.VMEM((_C,), jnp.float32),
        pltpu.VMEM_SHARED((_N_PAD,), jnp.float32),
        pltpu.SemaphoreType.DMA,
        pltpu.SemaphoreType.DMA,
        pltpu.SemaphoreType.DMA,
        pltpu.SemaphoreType.DMA,
        pltpu.SemaphoreType.DMA,
        pltpu.SemaphoreType.DMA,
    ],
)
def _scatter(scaled_hbm, edge_hbm, out_hbm,
             ang_t, ed0, ed1, ed2, ed3, row0, row1, val0, val1, step_sh,
             es0, es1, es2, es3, ss0, ss1):
    c = lax.axis_index("c")
    s = lax.axis_index("s")
    wid = s * _NC + c
    kbase = wid * _KPW
    ed = (ed0, ed1, ed2, ed3)
    es = (es0, es1, es2, es3)
    row = (row0, row1)
    val = (val0, val1)
    ss = (ss0, ss1)

    def _echunk(k):
        return edge_hbm.at[:, pl.ds(pl.multiple_of(k * _C, 128), _C)]

    for q in range(4):
        pltpu.async_copy(_echunk(kbase + q), ed[q], es[q])

    @plsc.parallel_loop(0, _C, step=16, unroll=8)
    def _zero(i):
        val0[pl.ds(i, 16)] = jnp.zeros((16,), jnp.float32)

    base = s * _SLICE
    for off in range(0, _SLICE - _C + 1, _C):
        pltpu.sync_copy(val0, step_sh.at[pl.ds(base + off, _C)])
    rem = _SLICE % _C
    if rem:
        pltpu.sync_copy(val0.at[pl.ds(0, rem)],
                        step_sh.at[pl.ds(base + _SLICE - rem, rem)])
    pltpu.sync_copy(scaled_hbm, ang_t)
    plsc.subcore_barrier()

    def _gather_chunk(edb, rowb, valb):
        @plsc.parallel_loop(0, _C, step=16, unroll=16)
        def _gather(i):
            idx = edb[1, pl.ds(i, 16)]
            valb[pl.ds(i, 16)] = plsc.load_gather(ang_t, [idx])
            rowb[pl.ds(i, 16)] = edb[0, pl.ds(i, 16)]

    def _step(j, e, v):
        pltpu.make_async_copy(_echunk(0), ed[e], es[e]).wait()

        @pl.when(j >= 2)
        def _():
            pltpu.make_async_copy(val[v], step_sh.at[row[v]], ss[v]).wait()

        _gather_chunk(ed[e], row[v], val[v])

        nk = jnp.minimum(kbase + j + 4, _NK - 1)
        pltpu.async_copy(_echunk(nk), ed[e], es[e])

        pltpu.async_copy(val[v], step_sh.at[row[v]], ss[v], add=True)

    def _quad(qq, carry):
        for t in range(4):
            _step(qq * 4 + t, t, t % 2)
        return carry

    lax.fori_loop(0, (_KPW - 1) // 4, _quad, 0)

    pltpu.make_async_copy(val0, step_sh.at[row0], ss0).wait()
    pltpu.make_async_copy(val1, step_sh.at[row1], ss1).wait()
    pltpu.make_async_copy(_echunk(0), ed0, es0).wait()
    _gather_chunk(ed0, row0, val0)
    pltpu.sync_copy(val0, step_sh.at[row0], add=True)
    pltpu.make_async_copy(_echunk(0), ed1, es1).wait()
    pltpu.make_async_copy(_echunk(0), ed2, es2).wait()
    pltpu.make_async_copy(_echunk(0), ed3, es3).wait()

    @pl.when(wid < _REM)
    def _rem():
        pltpu.sync_copy(_echunk(_KPW * _NW + wid), ed0)
        _gather_chunk(ed0, row0, val0)
        pltpu.sync_copy(val0, step_sh.at[row0], add=True)

    plsc.subcore_barrier()
    pltpu.sync_copy(step_sh.at[pl.ds(s * _SLICE, _SLICE)],
                    out_hbm.at[c, pl.ds(s * _SLICE, _SLICE)])


def _combine_body(a_ref, p0_ref, p1_ref, o_ref):
    o_ref[...] = a_ref[...] + p0_ref[...] + p1_ref[...]


def _combine(angles_r, p0, p1):
    return pl.pallas_call(
        _combine_body,
        out_shape=jax.ShapeDtypeStruct((_TROW, _TCOL), jnp.float32),
    )(angles_r, p0, p1)


def kernel(coeffs, edge_index, W0, b0, W1, W2, W3, b3):
    angles_r, scaled_r = _mlp(coeffs, W0, b0, W1, W2, W3, b3)
    partials = _scatter(scaled_r.reshape(_N_PAD)[:_N], edge_index)
    p = partials.reshape(_NC, _TROW, _TCOL)
    out = _combine(angles_r, p[0], p[1])
    return out.reshape(_N_PAD, 1)[:_N]


# speedup vs baseline: 1.1144x; 1.1144x over previous
import functools
import math

import jax
import jax.numpy as jnp
from jax import lax
from jax.experimental import pallas as pl
from jax.experimental.pallas import tpu as pltpu
from jax.experimental.pallas import tpu_sc as plsc

LN2 = math.log(2.0)

_N = 100000
_E = 6400000
_D = 128

_BLK = 4096
_NBLK = 25
_TROW = 400
_TCOL = 256
_N_PAD = _TROW * _TCOL
_NC, _NS = 2, 16
_NW = _NC * _NS
_SLICE = _N_PAD // _NS
_C = 2560
_NK = _E // _C
_KPW = _NK // _NW
_REM = _NK - _KPW * _NW


def _mlp_body(x_ref, w0_ref, b0_ref, w1_ref, w2_ref, w3_ref, b3_ref,
              ang_ref, scl_ref):
    dn = (((1,), (1,)), ((), ()))
    x = x_ref[...]
    h = jnp.maximum(
        lax.dot_general(x, w0_ref[...], dn, preferred_element_type=jnp.float32)
        + b0_ref[...], 0.0)
    h = jnp.maximum(
        lax.dot_general(h, w1_ref[...], dn, preferred_element_type=jnp.float32),
        0.0)
    h = jnp.maximum(
        lax.dot_general(h, w2_ref[...], dn, preferred_element_type=jnp.float32),
        0.0)
    a = (lax.dot_general(w3_ref[...], h, dn, preferred_element_type=jnp.float32)
         + b3_ref[0])
    a8 = jnp.concatenate(jnp.split(a, 16, axis=1), axis=0)
    ang_ref[...] = a8
    scl_ref[...] = a8 * LN2


def _mlp(coeffs, W0, b0, W1, W2, W3, b3):
    full = lambda i: (0, 0)
    return pl.pallas_call(
        _mlp_body,
        grid=(_NBLK,),
        in_specs=[
            pl.BlockSpec((_BLK, _D), lambda i: (i, 0)),
            pl.BlockSpec((_D, _D), full),
            pl.BlockSpec((1, _D), full),
            pl.BlockSpec((_D, _D), full),
            pl.BlockSpec((_D, _D), full),
            pl.BlockSpec((1, _D), full),
            pl.BlockSpec(memory_space=pltpu.SMEM),
        ],
        out_specs=[
            pl.BlockSpec((16, _TCOL), lambda i: (i, 0)),
            pl.BlockSpec((16, _TCOL), lambda i: (i, 0)),
        ],
        out_shape=[
            jax.ShapeDtypeStruct((_TROW, _TCOL), jnp.float32),
            jax.ShapeDtypeStruct((_TROW, _TCOL), jnp.float32),
        ],
    )(coeffs, W0, b0.reshape(1, _D), W1, W2, W3, b3)


@functools.partial(
    pl.kernel,
    out_type=jax.ShapeDtypeStruct((_NC, _N_PAD), jnp.float32),
    mesh=plsc.VectorSubcoreMesh(core_axis_name="c", subcore_axis_name="s"),
    compiler_params=pltpu.CompilerParams(needs_layout_passes=False),
    scratch_types=[
        pltpu.VMEM((_N,), jnp.float32),
        pltpu.VMEM((2, _C), jnp.int32),
        pltpu.VMEM((2, _C), jnp.int32),
        pltpu.VMEM((_C // 2,), jnp.int32),
        pltpu.VMEM((_C // 2,), jnp.int32),
        pltpu.VMEM((_C // 2,), jnp.int32),
        pltpu.VMEM((_C // 2,), jnp.int32),
        pltpu.VMEM((_C // 2,), jnp.float32),
        pltpu.VMEM((_C // 2,), jnp.float32),
        pltpu.VMEM((_C // 2,), jnp.float32),
        pltpu.VMEM((_C // 2,), jnp.float32),
        pltpu.VMEM((_SLICE // 2,), jnp.float32),
        pltpu.VMEM_SHARED((_N_PAD,), jnp.float32),
        pltpu.SemaphoreType.DMA,
        pltpu.SemaphoreType.DMA,
        pltpu.SemaphoreType.DMA,
        pltpu.SemaphoreType.DMA,
        pltpu.SemaphoreType.DMA,
        pltpu.SemaphoreType.DMA,
    ],
)
def _scatter(scaled_hbm, edge_hbm, out_hbm,
             ang_t, ed_a, ed_b, row_a0, row_a1, row_b0, row_b1,
             val_a0, val_a1, val_b0, val_b1, zbuf, step_sh,
             es_a, es_b, ss_a0, ss_a1, ss_b0, ss_b1):
    c = lax.axis_index("c")
    s = lax.axis_index("s")
    wid = s * _NC + c
    kbase = wid * _KPW

    @plsc.parallel_loop(0, _SLICE // 2, step=16, unroll=8)
    def _zero(i):
        zbuf[pl.ds(i, 16)] = jnp.zeros((16,), jnp.float32)

    pltpu.sync_copy(zbuf, step_sh.at[pl.ds(s * _SLICE, _SLICE // 2)])
    pltpu.sync_copy(zbuf,
                    step_sh.at[pl.ds(s * _SLICE + _SLICE // 2, _SLICE // 2)])
    pltpu.sync_copy(scaled_hbm, ang_t)
    plsc.subcore_barrier()

    def _echunk(k):
        return edge_hbm.at[:, pl.ds(pl.multiple_of(k * _C, 128), _C)]

    def _gather_chunk(edb, rows, vals):
        for kk in range(2):
            @plsc.parallel_loop(0, _C // 2, step=16, unroll=8)
            def _gather(i, kk=kk):
                idx = edb[1, pl.ds(kk * (_C // 2) + i, 16)]
                vals[kk][pl.ds(i, 16)] = plsc.load_gather(ang_t, [idx])
                rows[kk][pl.ds(i, 16)] = edb[0, pl.ds(kk * (_C // 2) + i, 16)]

    pltpu.async_copy(_echunk(kbase), ed_a, es_a)
    pltpu.async_copy(_echunk(kbase + 1), ed_b, es_b)

    def _half(j, edb, rows, vals, es, ss0, ss1):
        pltpu.make_async_copy(_echunk(0), edb, es).wait()

        @pl.when(j >= 2)
        def _():
            pltpu.make_async_copy(vals[0], step_sh.at[rows[0]], ss0).wait()
            pltpu.make_async_copy(vals[1], step_sh.at[rows[1]], ss1).wait()

        _gather_chunk(edb, rows, vals)

        nk = jnp.minimum(kbase + j + 2, _NK - 1)
        pltpu.async_copy(_echunk(nk), edb, es)

        pltpu.async_copy(vals[0], step_sh.at[rows[0]], ss0, add=True)
        pltpu.async_copy(vals[1], step_sh.at[rows[1]], ss1, add=True)

    def _pair(p, carry):
        _half(p * 2, ed_a, (row_a0, row_a1), (val_a0, val_a1),
              es_a, ss_a0, ss_a1)
        _half(p * 2 + 1, ed_b, (row_b0, row_b1), (val_b0, val_b1),
              es_b, ss_b0, ss_b1)
        return carry

    lax.fori_loop(0, _KPW // 2, _pair, 0)

    pltpu.make_async_copy(val_a0, step_sh.at[row_a0], ss_a0).wait()
    pltpu.make_async_copy(val_a1, step_sh.at[row_a1], ss_a1).wait()
    pltpu.make_async_copy(val_b0, step_sh.at[row_b0], ss_b0).wait()
    pltpu.make_async_copy(val_b1, step_sh.at[row_b1], ss_b1).wait()
    pltpu.make_async_copy(_echunk(0), ed_a, es_a).wait()
    pltpu.make_async_copy(_echunk(0), ed_b, es_b).wait()

    @pl.when(wid < _REM)
    def _rem():
        pltpu.sync_copy(_echunk(_KPW * _NW + wid), ed_a)
        _gather_chunk(ed_a, (row_a0, row_a1), (val_a0, val_a1))
        pltpu.sync_copy(val_a0, step_sh.at[row_a0], add=True)
        pltpu.sync_copy(val_a1, step_sh.at[row_a1], add=True)

    plsc.subcore_barrier()
    pltpu.sync_copy(step_sh.at[pl.ds(s * _SLICE, _SLICE)],
                    out_hbm.at[c, pl.ds(s * _SLICE, _SLICE)])


def _combine_body(a_ref, p0_ref, p1_ref, o_ref):
    o_ref[...] = a_ref[...] + p0_ref[...] + p1_ref[...]


def _combine(angles_r, p0, p1):
    return pl.pallas_call(
        _combine_body,
        out_shape=jax.ShapeDtypeStruct((_TROW, _TCOL), jnp.float32),
    )(angles_r, p0, p1)


def kernel(coeffs, edge_index, W0, b0, W1, W2, W3, b3):
    angles_r, scaled_r = _mlp(coeffs, W0, b0, W1, W2, W3, b3)
    partials = _scatter(scaled_r.reshape(_N_PAD)[:_N], edge_index)
    p = partials.reshape(_NC, _TROW, _TCOL)
    out = _combine(angles_r, p[0], p[1])
    return out.reshape(_N_PAD, 1)[:_N]


# speedup vs baseline: 1.1477x; 1.0298x over previous
import functools
import math

import jax
import jax.numpy as jnp
from jax import lax
from jax.experimental import pallas as pl
from jax.experimental.pallas import tpu as pltpu
from jax.experimental.pallas import tpu_sc as plsc

LN2 = math.log(2.0)

_N = 100000
_E = 6400000
_D = 128

_BLK = 8192
_NBLK = 13
_TROW = 416
_TCOL = 256
_N_PAD = _TROW * _TCOL
_NC, _NS = 2, 16
_NW = _NC * _NS
_SLICE = _N_PAD // _NS
_C = 2560
_NK = _E // _C
_KPW = _NK // _NW
_REM = _NK - _KPW * _NW


def _mlp_body(x_ref, w0_ref, b0_ref, w1_ref, w2_ref, w3_ref, b3_ref,
              ang_ref, scl_ref):
    dn = (((1,), (1,)), ((), ()))
    x = x_ref[...]
    h = jnp.maximum(
        lax.dot_general(x, w0_ref[...], dn, preferred_element_type=jnp.float32)
        + b0_ref[...], 0.0)
    h = jnp.maximum(
        lax.dot_general(h, w1_ref[...], dn, preferred_element_type=jnp.float32),
        0.0)
    h = jnp.maximum(
        lax.dot_general(h, w2_ref[...], dn, preferred_element_type=jnp.float32),
        0.0)
    a = (lax.dot_general(w3_ref[...], h, dn, preferred_element_type=jnp.float32)
         + b3_ref[0])
    a8 = jnp.concatenate(jnp.split(a, 32, axis=1), axis=0)
    ang_ref[...] = a8
    scl_ref[...] = a8 * LN2


def _mlp(coeffs, W0, b0, W1, W2, W3, b3):
    full = lambda i: (0, 0)
    return pl.pallas_call(
        _mlp_body,
        grid=(_NBLK,),
        in_specs=[
            pl.BlockSpec((_BLK, _D), lambda i: (i, 0)),
            pl.BlockSpec((_D, _D), full),
            pl.BlockSpec((1, _D), full),
            pl.BlockSpec((_D, _D), full),
            pl.BlockSpec((_D, _D), full),
            pl.BlockSpec((1, _D), full),
            pl.BlockSpec(memory_space=pltpu.SMEM),
        ],
        out_specs=[
            pl.BlockSpec((32, _TCOL), lambda i: (i, 0)),
            pl.BlockSpec((32, _TCOL), lambda i: (i, 0)),
        ],
        out_shape=[
            jax.ShapeDtypeStruct((_TROW, _TCOL), jnp.float32),
            jax.ShapeDtypeStruct((_TROW, _TCOL), jnp.float32),
        ],
    )(coeffs, W0, b0.reshape(1, _D), W1, W2, W3, b3)


@functools.partial(
    pl.kernel,
    out_type=jax.ShapeDtypeStruct((_NC, _N_PAD), jnp.float32),
    mesh=plsc.VectorSubcoreMesh(core_axis_name="c", subcore_axis_name="s"),
    compiler_params=pltpu.CompilerParams(needs_layout_passes=False),
    scratch_types=[
        pltpu.VMEM((_N,), jnp.float32),
        pltpu.VMEM((2, _C), jnp.int32),
        pltpu.VMEM((2, _C), jnp.int32),
        pltpu.VMEM((_C // 2,), jnp.int32),
        pltpu.VMEM((_C // 2,), jnp.int32),
        pltpu.VMEM((_C // 2,), jnp.int32),
        pltpu.VMEM((_C // 2,), jnp.int32),
        pltpu.VMEM((_C // 2,), jnp.float32),
        pltpu.VMEM((_C // 2,), jnp.float32),
        pltpu.VMEM((_C // 2,), jnp.float32),
        pltpu.VMEM((_C // 2,), jnp.float32),
        pltpu.VMEM((_SLICE // 2,), jnp.float32),
        pltpu.VMEM_SHARED((_N_PAD,), jnp.float32),
        pltpu.SemaphoreType.DMA,
        pltpu.SemaphoreType.DMA,
        pltpu.SemaphoreType.DMA,
        pltpu.SemaphoreType.DMA,
        pltpu.SemaphoreType.DMA,
        pltpu.SemaphoreType.DMA,
    ],
)
def _scatter(scaled_hbm, edge_hbm, out_hbm,
             ang_t, ed_a, ed_b, row_a0, row_a1, row_b0, row_b1,
             val_a0, val_a1, val_b0, val_b1, zbuf, step_sh,
             es_a, es_b, ss_a0, ss_a1, ss_b0, ss_b1):
    c = lax.axis_index("c")
    s = lax.axis_index("s")
    wid = s * _NC + c
    kbase = wid * _KPW

    @plsc.parallel_loop(0, _SLICE // 2, step=16, unroll=8)
    def _zero(i):
        zbuf[pl.ds(i, 16)] = jnp.zeros((16,), jnp.float32)

    pltpu.sync_copy(zbuf, step_sh.at[pl.ds(s * _SLICE, _SLICE // 2)])
    pltpu.sync_copy(zbuf,
                    step_sh.at[pl.ds(s * _SLICE + _SLICE // 2, _SLICE // 2)])
    pltpu.sync_copy(scaled_hbm, ang_t)
    plsc.subcore_barrier()

    def _echunk(k):
        return edge_hbm.at[:, pl.ds(pl.multiple_of(k * _C, 128), _C)]

    def _gather_chunk(edb, rows, vals):
        for kk in range(2):
            @plsc.parallel_loop(0, _C // 2, step=16, unroll=8)
            def _gather(i, kk=kk):
                idx = edb[1, pl.ds(kk * (_C // 2) + i, 16)]
                vals[kk][pl.ds(i, 16)] = plsc.load_gather(ang_t, [idx])
                rows[kk][pl.ds(i, 16)] = edb[0, pl.ds(kk * (_C // 2) + i, 16)]

    pltpu.async_copy(_echunk(kbase), ed_a, es_a)
    pltpu.async_copy(_echunk(kbase + 1), ed_b, es_b)

    def _half(j, edb, rows, vals, es, ss0, ss1):
        pltpu.make_async_copy(_echunk(0), edb, es).wait()

        @pl.when(j >= 2)
        def _():
            pltpu.make_async_copy(vals[0], step_sh.at[rows[0]], ss0).wait()
            pltpu.make_async_copy(vals[1], step_sh.at[rows[1]], ss1).wait()

        _gather_chunk(edb, rows, vals)

        nk = jnp.minimum(kbase + j + 2, _NK - 1)
        pltpu.async_copy(_echunk(nk), edb, es)

        pltpu.async_copy(vals[0], step_sh.at[rows[0]], ss0, add=True)
        pltpu.async_copy(vals[1], step_sh.at[rows[1]], ss1, add=True)

    def _pair(p, carry):
        _half(p * 2, ed_a, (row_a0, row_a1), (val_a0, val_a1),
              es_a, ss_a0, ss_a1)
        _half(p * 2 + 1, ed_b, (row_b0, row_b1), (val_b0, val_b1),
              es_b, ss_b0, ss_b1)
        return carry

    lax.fori_loop(0, _KPW // 2, _pair, 0)

    pltpu.make_async_copy(val_a0, step_sh.at[row_a0], ss_a0).wait()
    pltpu.make_async_copy(val_a1, step_sh.at[row_a1], ss_a1).wait()
    pltpu.make_async_copy(val_b0, step_sh.at[row_b0], ss_b0).wait()
    pltpu.make_async_copy(val_b1, step_sh.at[row_b1], ss_b1).wait()
    pltpu.make_async_copy(_echunk(0), ed_a, es_a).wait()
    pltpu.make_async_copy(_echunk(0), ed_b, es_b).wait()

    @pl.when(wid < _REM)
    def _rem():
        pltpu.sync_copy(_echunk(_KPW * _NW + wid), ed_a)
        _gather_chunk(ed_a, (row_a0, row_a1), (val_a0, val_a1))
        pltpu.sync_copy(val_a0, step_sh.at[row_a0], add=True)
        pltpu.sync_copy(val_a1, step_sh.at[row_a1], add=True)

    plsc.subcore_barrier()
    pltpu.sync_copy(step_sh.at[pl.ds(s * _SLICE, _SLICE)],
                    out_hbm.at[c, pl.ds(s * _SLICE, _SLICE)])


def _combine_body(a_ref, p0_ref, p1_ref, o_ref):
    o_ref[...] = a_ref[...] + p0_ref[...] + p1_ref[...]


def _combine(angles_r, p0, p1):
    return pl.pallas_call(
        _combine_body,
        out_shape=jax.ShapeDtypeStruct((_TROW, _TCOL), jnp.float32),
    )(angles_r, p0, p1)


def kernel(coeffs, edge_index, W0, b0, W1, W2, W3, b3):
    angles_r, scaled_r = _mlp(coeffs, W0, b0, W1, W2, W3, b3)
    partials = _scatter(scaled_r.reshape(_N_PAD)[:_N], edge_index)
    p = partials.reshape(_NC, _TROW, _TCOL)
    out = _combine(angles_r, p[0], p[1])
    return out.reshape(_N_PAD, 1)[:_N]
